# baseline (speedup 1.0000x reference)
import functools

import jax
import jax.numpy as jnp
from jax import lax
from jax.experimental import pallas as pl
from jax.experimental.pallas import tpu as pltpu
from jax.experimental.pallas import tpu_sc as plsc

N = 10000
E = 320000
D = 128
DOUT = 64
DA = 144
BR = 400
GRID = N // BR
C = 128
NW = 32
EPW = E // NW
TAIL = 16
CHW = (EPW - TAIL) // C
RPT = 624
ZCH = 104
REM = N - RPT * 16
LEAK = 0.2


def _leaky(z):
    return jnp.where(z > 0, z, z * LEAK)


def _scores(y, A):
    S = jnp.dot(y, A, preferred_element_type=jnp.float32)
    x1 = S[:, 0:1]
    s2 = S[:, 1:2]
    h1 = S[:, 2:3]
    w2 = jnp.exp(_leaky(x1 + s2))
    ci = lax.broadcasted_iota(jnp.int32, (BR, 8), 1)
    return jnp.where(ci == 0, x1, jnp.where(ci == 1, w2,
                     jnp.where(ci == 2, h1, 0.0)))


def _write_haug(haug_ref, y, h1):
    haug_ref[:, pl.ds(0, D)] = y
    ci = lax.broadcasted_iota(jnp.int32, (BR, 16), 1)
    haug_ref[:, pl.ds(D, 16)] = jnp.where(ci == 0, 1.0,
                                          jnp.where(ci == 1, h1, 0.0))


def _prep0_body(x_ref, W1_ref, b1_ref, Wh_ref, bh_ref, A_ref,
                y_ref, haug_ref, sc_ref):
    t = jnp.maximum(
        jnp.dot(x_ref[...], W1_ref[...], preferred_element_type=jnp.float32)
        + b1_ref[...], 0.0)
    y = jnp.dot(t, Wh_ref[...], preferred_element_type=jnp.float32) + bh_ref[...]
    y_ref[...] = y
    sc = _scores(y, A_ref[...])
    sc_ref[...] = sc
    _write_haug(haug_ref, y, sc[:, 2:3])


def _combine(acc_ref, y_ref, sc_ref):
    acc = acc_ref[0] + acc_ref[1]
    aggr = acc[:, 0:D]
    div = acc[:, D:D + 1]
    w2 = sc_ref[...][:, 1:2]
    y = y_ref[...]
    z = (aggr + w2 * y) / (div + w2)
    return jnp.where(z > 0, z, jnp.exp(jnp.minimum(z, 0.0)) - 1.0)


def _combine_prep_body(acc_ref, y_ref, sc_ref, Wh_ref, bh_ref, A_ref,
                       y2_ref, haug_ref, sc2_ref):
    z = _combine(acc_ref, y_ref, sc_ref)
    y2 = jnp.dot(z, Wh_ref[...], preferred_element_type=jnp.float32) + bh_ref[...]
    y2_ref[...] = y2
    sc2 = _scores(y2, A_ref[...])
    sc2_ref[...] = sc2
    _write_haug(haug_ref, y2, sc2[:, 2:3])


def _final_body(acc_ref, y_ref, sc_ref, W2_ref, b2_ref, out_ref):
    z = _combine(acc_ref, y_ref, sc_ref)
    out_ref[...] = (
        jnp.dot(z, W2_ref[...], preferred_element_type=jnp.float32) + b2_ref[...])


_ROWB = lambda w: pl.BlockSpec((BR, w), lambda i: (i, 0))
_BCAST = lambda r, c: pl.BlockSpec((r, c), lambda i: (0, 0))
_ACCB = pl.BlockSpec((2, BR, DA), lambda i: (0, i, 0))

_PREP_OUT = (
    [jax.ShapeDtypeStruct((N, D), jnp.float32),
     jax.ShapeDtypeStruct((N, DA), jnp.float32),
     jax.ShapeDtypeStruct((N, 8), jnp.float32)],
    [_ROWB(D), _ROWB(DA), _ROWB(8)],
)


def _tc_prep0(x, W1, b1, Wh, bh, A):
    return pl.pallas_call(
        _prep0_body,
        grid=(GRID,),
        in_specs=[_ROWB(D), _BCAST(D, D), _BCAST(1, D), _BCAST(D, D),
                  _BCAST(1, D), _BCAST(D, 8)],
        out_specs=_PREP_OUT[1],
        out_shape=_PREP_OUT[0],
    )(x, W1, b1, Wh, bh, A)


def _tc_combine_prep(acc, y, sc, Wh, bh, A):
    return pl.pallas_call(
        _combine_prep_body,
        grid=(GRID,),
        in_specs=[_ACCB, _ROWB(D), _ROWB(8), _BCAST(D, D), _BCAST(1, D),
                  _BCAST(D, 8)],
        out_specs=_PREP_OUT[1],
        out_shape=_PREP_OUT[0],
    )(acc, y, sc, Wh, bh, A)


def _tc_final(acc, y, sc, W2, b2):
    return pl.pallas_call(
        _final_body,
        grid=(GRID,),
        in_specs=[_ACCB, _ROWB(D), _ROWB(8), _BCAST(D, DOUT),
                  _BCAST(1, DOUT)],
        out_specs=_ROWB(DOUT),
        out_shape=jax.ShapeDtypeStruct((N, DOUT), jnp.float32),
    )(acc, y, sc, W2, b2)


def _sc_body(s_hbm, t_hbm, haug_hbm, x1_hbm, h1_hbm, out_hbm,
             s0, t0, w0, xs0, hs0, rows0, s1, t1, w1, xs1, hs1, rows1,
             st_loc, tt_loc, wt_loc, acc, sem0, sem1):
    c = lax.axis_index("c")
    s = lax.axis_index("s")
    wid = s * 2 + c
    s_loc = (s0, s1)
    t_loc = (t0, t1)
    w_loc = (w0, w1)
    xs_loc = (xs0, xs1)
    hs_loc = (hs0, hs1)
    rows = (rows0, rows1)
    sem = (sem0, sem1)

    @pl.loop(0, C)
    def _zero(e):
        for k in range(DA // 16):
            rows0[e, pl.ds(k * 16, 16)] = jnp.zeros((16,), jnp.float32)

    row0 = s * RPT
    for m in range(RPT // ZCH):
        pltpu.sync_copy(rows0.at[pl.ds(0, ZCH)],
                        acc.at[pl.ds(row0 + m * ZCH, ZCH)])

    @pl.when(s == 15)
    def _zero_rem():
        pltpu.sync_copy(rows0.at[pl.ds(0, REM)], acc.at[pl.ds(RPT * 16, REM)])

    plsc.subcore_barrier()

    base_w = wid * EPW

    def stage_idx(j, b):
        pltpu.sync_copy(s_hbm.at[pl.ds(base_w + j * C, C)], s_loc[b])
        pltpu.sync_copy(t_hbm.at[pl.ds(base_w + j * C, C)], t_loc[b])
        pltpu.async_copy(haug_hbm.at[t_loc[b]], rows[b], sem[b])
        pltpu.async_copy(x1_hbm.at[s_loc[b]], xs_loc[b], sem[b])
        pltpu.async_copy(h1_hbm.at[t_loc[b]], hs_loc[b], sem[b])

    def process(b, nrows):
        pltpu.make_async_copy(haug_hbm.at[t_loc[b]], rows[b], sem[b]).wait()
        pltpu.make_async_copy(x1_hbm.at[s_loc[b]], xs_loc[b], sem[b]).wait()
        pltpu.make_async_copy(h1_hbm.at[t_loc[b]], hs_loc[b], sem[b]).wait()
        for g in range(nrows // 16):
            sl16 = pl.ds(g * 16, 16)
            z = xs_loc[b][sl16] + hs_loc[b][sl16]
            w_loc[b][sl16] = jnp.exp(_leaky(z))

        @pl.loop(0, nrows)
        def _scale(e):
            wv = plsc.load_gather(w_loc[b], [jnp.full((16,), e, jnp.int32)])
            for k in range(DA // 16):
                sl = pl.ds(k * 16, 16)
                rows[b][e, sl] = rows[b][e, sl] * wv

    def pair_body(p, carry):
        for b in (0, 1):
            stage_idx(2 * p + b, b)
            process(b, C)
            pltpu.sync_copy(rows[b], acc.at[s_loc[b]], add=True)
        return carry

    lax.fori_loop(0, CHW // 2, pair_body, 0)

    tb = base_w + CHW * C
    pltpu.sync_copy(s_hbm.at[pl.ds(tb, TAIL)], st_loc)
    pltpu.sync_copy(t_hbm.at[pl.ds(tb, TAIL)], tt_loc)
    pltpu.async_copy(haug_hbm.at[tt_loc], rows0.at[pl.ds(0, TAIL)], sem0)
    pltpu.async_copy(x1_hbm.at[st_loc], xs0.at[pl.ds(0, TAIL)], sem0)
    pltpu.async_copy(h1_hbm.at[tt_loc], hs0.at[pl.ds(0, TAIL)], sem0)
    pltpu.make_async_copy(haug_hbm.at[tt_loc], rows0.at[pl.ds(0, TAIL)], sem0).wait()
    pltpu.make_async_copy(x1_hbm.at[st_loc], xs0.at[pl.ds(0, TAIL)], sem0).wait()
    pltpu.make_async_copy(h1_hbm.at[tt_loc], hs0.at[pl.ds(0, TAIL)], sem0).wait()
    zt = xs0[pl.ds(0, TAIL)] + hs0[pl.ds(0, TAIL)]
    wt_loc[...] = jnp.exp(_leaky(zt))

    @pl.loop(0, TAIL)
    def _scale_tail(e):
        wv = plsc.load_gather(wt_loc, [jnp.full((16,), e, jnp.int32)])
        for k in range(DA // 16):
            sl = pl.ds(k * 16, 16)
            rows0[e, sl] = rows0[e, sl] * wv

    pltpu.sync_copy(rows0.at[pl.ds(0, TAIL)], acc.at[st_loc], add=True)
    plsc.subcore_barrier()
    for m in range(RPT // ZCH):
        sl = pl.ds(row0 + m * ZCH, ZCH)
        pltpu.sync_copy(acc.at[sl], out_hbm.at[c, sl])

    @pl.when(s == 15)
    def _out_rem():
        sl = pl.ds(RPT * 16, REM)
        pltpu.sync_copy(acc.at[sl], out_hbm.at[c, sl])


def _sc_edge_pass(s_idx, t_idx, haug, x1, h1):
    mesh = plsc.VectorSubcoreMesh(core_axis_name="c", subcore_axis_name="s")
    buf = [
        pltpu.VMEM((C,), jnp.int32),
        pltpu.VMEM((C,), jnp.int32),
        pltpu.VMEM((C,), jnp.float32),
        pltpu.VMEM((C,), jnp.float32),
        pltpu.VMEM((C,), jnp.float32),
        pltpu.VMEM((C, DA), jnp.float32),
    ]
    return pl.kernel(
        _sc_body,
        out_type=jax.ShapeDtypeStruct((2, N, DA), jnp.float32),
        mesh=mesh,
        compiler_params=pltpu.CompilerParams(needs_layout_passes=False,
                                             use_tc_tiling_on_sc=False),
        scratch_types=buf + buf + [
            pltpu.VMEM((TAIL,), jnp.int32),
            pltpu.VMEM((TAIL,), jnp.int32),
            pltpu.VMEM((TAIL,), jnp.float32),
            pltpu.VMEM_SHARED((N, DA), jnp.float32),
            pltpu.SemaphoreType.DMA,
            pltpu.SemaphoreType.DMA,
        ],
    )(s_idx, t_idx, haug, x1, h1)


def _amat(a1v, a2v, a2o):
    A = jnp.zeros((D, 8), jnp.float32)
    return A.at[:, 0].set(a1v).at[:, 1].set(a2v).at[:, 2].set(a2o)


def kernel(x_user, x_item, edge_index_ui, edge_index_iu, W1_user, b1_user,
           W1_item, b1_item, Wh, bh, a1, a2, W2, b2):
    su = edge_index_ui[0].astype(jnp.int32)
    tu = edge_index_ui[1].astype(jnp.int32)
    si = edge_index_iu[0].astype(jnp.int32)
    ti = edge_index_iu[1].astype(jnp.int32)

    b1u = b1_user.reshape(1, D)
    b1i = b1_item.reshape(1, D)
    bh0 = bh[0].reshape(1, D)
    bh1 = bh[1].reshape(1, D)
    b2r = b2.reshape(1, DOUT)

    A_u0 = _amat(a1[0, 0], a2[0, 0], a2[0, 1])
    A_i0 = _amat(a1[0, 1], a2[0, 1], a2[0, 0])
    A_u1 = _amat(a1[1, 0], a2[1, 0], a2[1, 1])
    A_i1 = _amat(a1[1, 1], a2[1, 1], a2[1, 0])

    yu0, haug_u0, scu0 = _tc_prep0(x_user, W1_user, b1u, Wh[0], bh0, A_u0)
    yi0, haug_i0, sci0 = _tc_prep0(x_item, W1_item, b1i, Wh[0], bh0, A_i0)

    acc_u0 = _sc_edge_pass(su, tu, haug_i0, jnp.copy(scu0[:, 0]),
                           jnp.copy(sci0[:, 2]))
    acc_i0 = _sc_edge_pass(si, ti, haug_u0, jnp.copy(sci0[:, 0]),
                           jnp.copy(scu0[:, 2]))

    yu1, _, scu1 = _tc_combine_prep(acc_u0, yu0, scu0, Wh[1], bh1, A_u1)
    yi1, haug_i1, sci1 = _tc_combine_prep(acc_i0, yi0, sci0, Wh[1], bh1, A_i1)

    acc_u1 = _sc_edge_pass(su, tu, haug_i1, jnp.copy(scu1[:, 0]),
                           jnp.copy(sci1[:, 2]))

    return _tc_final(acc_u1, yu1, scu1, W2, b2r)


# speedup vs baseline: 1.3799x; 1.3799x over previous
import functools

import jax
import jax.numpy as jnp
from jax import lax
from jax.experimental import pallas as pl
from jax.experimental.pallas import tpu as pltpu
from jax.experimental.pallas import tpu_sc as plsc

N = 10000
E = 320000
D = 128
DOUT = 64
DA = 144
BR = 400
GRID = N // BR
C = 128
NW = 32
EPW = E // NW
TAIL = 16
CHW = (EPW - TAIL) // C
RPT = 624
ZCH = 104
REM = N - RPT * 16
LEAK = 0.2


def _leaky(z):
    return jnp.where(z > 0, z, z * LEAK)


def _scores(y, A):
    S = jnp.dot(y, A, preferred_element_type=jnp.float32)
    x1 = S[:, 0:1]
    s2 = S[:, 1:2]
    h1 = S[:, 2:3]
    w2 = jnp.exp(_leaky(x1 + s2))
    ci = lax.broadcasted_iota(jnp.int32, (BR, 8), 1)
    return jnp.where(ci == 0, x1, jnp.where(ci == 1, w2,
                     jnp.where(ci == 2, h1, 0.0)))


def _write_haug(haug_ref, y, h1):
    haug_ref[:, pl.ds(0, D)] = y
    ci = lax.broadcasted_iota(jnp.int32, (BR, 16), 1)
    haug_ref[:, pl.ds(D, 16)] = jnp.where(ci == 0, 1.0,
                                          jnp.where(ci == 1, h1, 0.0))


def _prep0_body(x_ref, W1_ref, b1_ref, Wh_ref, bh_ref, A_ref,
                y_ref, haug_ref, sc_ref):
    t = jnp.maximum(
        jnp.dot(x_ref[...], W1_ref[...], preferred_element_type=jnp.float32)
        + b1_ref[...], 0.0)
    y = jnp.dot(t, Wh_ref[...], preferred_element_type=jnp.float32) + bh_ref[...]
    y_ref[...] = y
    sc = _scores(y, A_ref[...])
    sc_ref[...] = sc
    _write_haug(haug_ref, y, sc[:, 2:3])


def _combine(acc_ref, y_ref, sc_ref):
    acc = acc_ref[0] + acc_ref[1]
    aggr = acc[:, 0:D]
    div = acc[:, D:D + 1]
    w2 = sc_ref[...][:, 1:2]
    y = y_ref[...]
    z = (aggr + w2 * y) / (div + w2)
    return jnp.where(z > 0, z, jnp.exp(jnp.minimum(z, 0.0)) - 1.0)


def _combine_prep_body(acc_ref, y_ref, sc_ref, Wh_ref, bh_ref, A_ref,
                       y2_ref, haug_ref, sc2_ref):
    z = _combine(acc_ref, y_ref, sc_ref)
    y2 = jnp.dot(z, Wh_ref[...], preferred_element_type=jnp.float32) + bh_ref[...]
    y2_ref[...] = y2
    sc2 = _scores(y2, A_ref[...])
    sc2_ref[...] = sc2
    _write_haug(haug_ref, y2, sc2[:, 2:3])


def _final_body(acc_ref, y_ref, sc_ref, W2_ref, b2_ref, out_ref):
    z = _combine(acc_ref, y_ref, sc_ref)
    out_ref[...] = (
        jnp.dot(z, W2_ref[...], preferred_element_type=jnp.float32) + b2_ref[...])


_ROWB = lambda w: pl.BlockSpec((BR, w), lambda i: (i, 0))
_BCAST = lambda r, c: pl.BlockSpec((r, c), lambda i: (0, 0))
_ACCB = pl.BlockSpec((2, BR, DA), lambda i: (0, i, 0))

_PREP_OUT = (
    [jax.ShapeDtypeStruct((N, D), jnp.float32),
     jax.ShapeDtypeStruct((N, DA), jnp.float32),
     jax.ShapeDtypeStruct((N, 8), jnp.float32)],
    [_ROWB(D), _ROWB(DA), _ROWB(8)],
)


def _tc_prep0(x, W1, b1, Wh, bh, A):
    return pl.pallas_call(
        _prep0_body,
        grid=(GRID,),
        in_specs=[_ROWB(D), _BCAST(D, D), _BCAST(1, D), _BCAST(D, D),
                  _BCAST(1, D), _BCAST(D, 8)],
        out_specs=_PREP_OUT[1],
        out_shape=_PREP_OUT[0],
    )(x, W1, b1, Wh, bh, A)


def _tc_combine_prep(acc, y, sc, Wh, bh, A):
    return pl.pallas_call(
        _combine_prep_body,
        grid=(GRID,),
        in_specs=[_ACCB, _ROWB(D), _ROWB(8), _BCAST(D, D), _BCAST(1, D),
                  _BCAST(D, 8)],
        out_specs=_PREP_OUT[1],
        out_shape=_PREP_OUT[0],
    )(acc, y, sc, Wh, bh, A)


def _tc_final(acc, y, sc, W2, b2):
    return pl.pallas_call(
        _final_body,
        grid=(GRID,),
        in_specs=[_ACCB, _ROWB(D), _ROWB(8), _BCAST(D, DOUT),
                  _BCAST(1, DOUT)],
        out_specs=_ROWB(DOUT),
        out_shape=jax.ShapeDtypeStruct((N, DOUT), jnp.float32),
    )(acc, y, sc, W2, b2)


def _sc_body(s_hbm, t_hbm, haug_hbm, x1_hbm, h1_hbm, out_hbm,
             s0, t0, w0, xs0, hs0, rows0, s1, t1, w1, xs1, hs1, rows1,
             st_loc, tt_loc, wt_loc, acc, sem0, sem1):
    c = lax.axis_index("c")
    s = lax.axis_index("s")
    wid = s * 2 + c
    s_loc = (s0, s1)
    t_loc = (t0, t1)
    w_loc = (w0, w1)
    xs_loc = (xs0, xs1)
    hs_loc = (hs0, hs1)
    rows = (rows0, rows1)
    sem = (sem0, sem1)

    @pl.loop(0, C)
    def _zero(e):
        for k in range(DA // 16):
            rows0[e, pl.ds(k * 16, 16)] = jnp.zeros((16,), jnp.float32)

    row0 = s * RPT
    for m in range(RPT // ZCH):
        pltpu.sync_copy(rows0.at[pl.ds(0, ZCH)],
                        acc.at[pl.ds(row0 + m * ZCH, ZCH)])

    @pl.when(s == 15)
    def _zero_rem():
        pltpu.sync_copy(rows0.at[pl.ds(0, REM)], acc.at[pl.ds(RPT * 16, REM)])

    plsc.subcore_barrier()

    base_w = wid * EPW

    def stage_idx(j, b):
        pltpu.sync_copy(s_hbm.at[pl.ds(base_w + j * C, C)], s_loc[b])
        pltpu.sync_copy(t_hbm.at[pl.ds(base_w + j * C, C)], t_loc[b])
        pltpu.async_copy(haug_hbm.at[t_loc[b]], rows[b], sem[b])
        pltpu.async_copy(x1_hbm.at[s_loc[b]], xs_loc[b], sem[b])
        pltpu.async_copy(h1_hbm.at[t_loc[b]], hs_loc[b], sem[b])

    def process(b, nrows):
        pltpu.make_async_copy(haug_hbm.at[t_loc[b]], rows[b], sem[b]).wait()
        pltpu.make_async_copy(x1_hbm.at[s_loc[b]], xs_loc[b], sem[b]).wait()
        pltpu.make_async_copy(h1_hbm.at[t_loc[b]], hs_loc[b], sem[b]).wait()
        for g in range(nrows // 16):
            sl16 = pl.ds(g * 16, 16)
            z = xs_loc[b][sl16] + hs_loc[b][sl16]
            w_loc[b][sl16] = jnp.exp(_leaky(z))

        @pl.loop(0, nrows)
        def _scale(e):
            wv = plsc.load_gather(w_loc[b], [jnp.full((16,), e, jnp.int32)])
            for k in range(DA // 16):
                sl = pl.ds(k * 16, 16)
                rows[b][e, sl] = rows[b][e, sl] * wv

    stage_idx(0, 0)
    stage_idx(1, 1)

    def pair_body(p, carry):
        for b in (0, 1):
            process(b, C)
            pltpu.sync_copy(rows[b], acc.at[s_loc[b]], add=True)

            @pl.when(p < CHW // 2 - 1)
            def _prefetch():
                stage_idx(2 * p + b + 2, b)
        return carry

    lax.fori_loop(0, CHW // 2, pair_body, 0)

    tb = base_w + CHW * C
    pltpu.sync_copy(s_hbm.at[pl.ds(tb, TAIL)], st_loc)
    pltpu.sync_copy(t_hbm.at[pl.ds(tb, TAIL)], tt_loc)
    pltpu.async_copy(haug_hbm.at[tt_loc], rows0.at[pl.ds(0, TAIL)], sem0)
    pltpu.async_copy(x1_hbm.at[st_loc], xs0.at[pl.ds(0, TAIL)], sem0)
    pltpu.async_copy(h1_hbm.at[tt_loc], hs0.at[pl.ds(0, TAIL)], sem0)
    pltpu.make_async_copy(haug_hbm.at[tt_loc], rows0.at[pl.ds(0, TAIL)], sem0).wait()
    pltpu.make_async_copy(x1_hbm.at[st_loc], xs0.at[pl.ds(0, TAIL)], sem0).wait()
    pltpu.make_async_copy(h1_hbm.at[tt_loc], hs0.at[pl.ds(0, TAIL)], sem0).wait()
    zt = xs0[pl.ds(0, TAIL)] + hs0[pl.ds(0, TAIL)]
    wt_loc[...] = jnp.exp(_leaky(zt))

    @pl.loop(0, TAIL)
    def _scale_tail(e):
        wv = plsc.load_gather(wt_loc, [jnp.full((16,), e, jnp.int32)])
        for k in range(DA // 16):
            sl = pl.ds(k * 16, 16)
            rows0[e, sl] = rows0[e, sl] * wv

    pltpu.sync_copy(rows0.at[pl.ds(0, TAIL)], acc.at[st_loc], add=True)
    plsc.subcore_barrier()
    for m in range(RPT // ZCH):
        sl = pl.ds(row0 + m * ZCH, ZCH)
        pltpu.sync_copy(acc.at[sl], out_hbm.at[c, sl])

    @pl.when(s == 15)
    def _out_rem():
        sl = pl.ds(RPT * 16, REM)
        pltpu.sync_copy(acc.at[sl], out_hbm.at[c, sl])


def _sc_edge_pass(s_idx, t_idx, haug, x1, h1):
    mesh = plsc.VectorSubcoreMesh(core_axis_name="c", subcore_axis_name="s")
    buf = [
        pltpu.VMEM((C,), jnp.int32),
        pltpu.VMEM((C,), jnp.int32),
        pltpu.VMEM((C,), jnp.float32),
        pltpu.VMEM((C,), jnp.float32),
        pltpu.VMEM((C,), jnp.float32),
        pltpu.VMEM((C, DA), jnp.float32),
    ]
    return pl.kernel(
        _sc_body,
        out_type=jax.ShapeDtypeStruct((2, N, DA), jnp.float32),
        mesh=mesh,
        compiler_params=pltpu.CompilerParams(needs_layout_passes=False,
                                             use_tc_tiling_on_sc=False),
        scratch_types=buf + buf + [
            pltpu.VMEM((TAIL,), jnp.int32),
            pltpu.VMEM((TAIL,), jnp.int32),
            pltpu.VMEM((TAIL,), jnp.float32),
            pltpu.VMEM_SHARED((N, DA), jnp.float32),
            pltpu.SemaphoreType.DMA,
            pltpu.SemaphoreType.DMA,
        ],
    )(s_idx, t_idx, haug, x1, h1)


def _amat(a1v, a2v, a2o):
    A = jnp.zeros((D, 8), jnp.float32)
    return A.at[:, 0].set(a1v).at[:, 1].set(a2v).at[:, 2].set(a2o)


def kernel(x_user, x_item, edge_index_ui, edge_index_iu, W1_user, b1_user,
           W1_item, b1_item, Wh, bh, a1, a2, W2, b2):
    su = edge_index_ui[0].astype(jnp.int32)
    tu = edge_index_ui[1].astype(jnp.int32)
    si = edge_index_iu[0].astype(jnp.int32)
    ti = edge_index_iu[1].astype(jnp.int32)

    b1u = b1_user.reshape(1, D)
    b1i = b1_item.reshape(1, D)
    bh0 = bh[0].reshape(1, D)
    bh1 = bh[1].reshape(1, D)
    b2r = b2.reshape(1, DOUT)

    A_u0 = _amat(a1[0, 0], a2[0, 0], a2[0, 1])
    A_i0 = _amat(a1[0, 1], a2[0, 1], a2[0, 0])
    A_u1 = _amat(a1[1, 0], a2[1, 0], a2[1, 1])
    A_i1 = _amat(a1[1, 1], a2[1, 1], a2[1, 0])

    yu0, haug_u0, scu0 = _tc_prep0(x_user, W1_user, b1u, Wh[0], bh0, A_u0)
    yi0, haug_i0, sci0 = _tc_prep0(x_item, W1_item, b1i, Wh[0], bh0, A_i0)

    acc_u0 = _sc_edge_pass(su, tu, haug_i0, jnp.copy(scu0[:, 0]),
                           jnp.copy(sci0[:, 2]))
    acc_i0 = _sc_edge_pass(si, ti, haug_u0, jnp.copy(sci0[:, 0]),
                           jnp.copy(scu0[:, 2]))

    yu1, _, scu1 = _tc_combine_prep(acc_u0, yu0, scu0, Wh[1], bh1, A_u1)
    yi1, haug_i1, sci1 = _tc_combine_prep(acc_i0, yi0, sci0, Wh[1], bh1, A_i1)

    acc_u1 = _sc_edge_pass(su, tu, haug_i1, jnp.copy(scu1[:, 0]),
                           jnp.copy(sci1[:, 2]))

    return _tc_final(acc_u1, yu1, scu1, W2, b2r)


# speedup vs baseline: 1.3977x; 1.0129x over previous
import functools

import jax
import jax.numpy as jnp
from jax import lax
from jax.experimental import pallas as pl
from jax.experimental.pallas import tpu as pltpu
from jax.experimental.pallas import tpu_sc as plsc

N = 10000
E = 320000
D = 128
DOUT = 64
DA = 144
BR = 400
GRID = N // BR
C = 128
NW = 32
EPW = E // NW
TAIL = 16
CHW = (EPW - TAIL) // C
RPT = 624
ZCH = 104
REM = N - RPT * 16
LEAK = 0.2


def _leaky(z):
    return jnp.where(z > 0, z, z * LEAK)


def _scores(y, A):
    S = jnp.dot(y, A, preferred_element_type=jnp.float32)
    x1 = S[:, 0:1]
    s2 = S[:, 1:2]
    h1 = S[:, 2:3]
    w2 = jnp.exp(_leaky(x1 + s2))
    ci = lax.broadcasted_iota(jnp.int32, (BR, 8), 1)
    return jnp.where(ci == 0, x1, jnp.where(ci == 1, w2,
                     jnp.where(ci == 2, h1, 0.0)))


def _write_haug(haug_ref, y, h1):
    haug_ref[:, pl.ds(0, D)] = y
    ci = lax.broadcasted_iota(jnp.int32, (BR, 16), 1)
    haug_ref[:, pl.ds(D, 16)] = jnp.where(ci == 0, 1.0,
                                          jnp.where(ci == 1, h1, 0.0))


def _prep0_body(x_ref, W1_ref, b1_ref, Wh_ref, bh_ref, A_ref,
                y_ref, haug_ref, sc_ref):
    t = jnp.maximum(
        jnp.dot(x_ref[...], W1_ref[...], preferred_element_type=jnp.float32)
        + b1_ref[...], 0.0)
    y = jnp.dot(t, Wh_ref[...], preferred_element_type=jnp.float32) + bh_ref[...]
    y_ref[...] = y
    sc = _scores(y, A_ref[...])
    sc_ref[...] = sc
    _write_haug(haug_ref, y, sc[:, 2:3])


def _combine(acc_ref, y_ref, sc_ref):
    acc = acc_ref[0] + acc_ref[1]
    aggr = acc[:, 0:D]
    div = acc[:, D:D + 1]
    w2 = sc_ref[...][:, 1:2]
    y = y_ref[...]
    z = (aggr + w2 * y) / (div + w2)
    return jnp.where(z > 0, z, jnp.exp(jnp.minimum(z, 0.0)) - 1.0)


def _combine_prep_body(acc_ref, y_ref, sc_ref, Wh_ref, bh_ref, A_ref,
                       y2_ref, haug_ref, sc2_ref):
    z = _combine(acc_ref, y_ref, sc_ref)
    y2 = jnp.dot(z, Wh_ref[...], preferred_element_type=jnp.float32) + bh_ref[...]
    y2_ref[...] = y2
    sc2 = _scores(y2, A_ref[...])
    sc2_ref[...] = sc2
    _write_haug(haug_ref, y2, sc2[:, 2:3])


def _final_body(acc_ref, y_ref, sc_ref, W2_ref, b2_ref, out_ref):
    z = _combine(acc_ref, y_ref, sc_ref)
    out_ref[...] = (
        jnp.dot(z, W2_ref[...], preferred_element_type=jnp.float32) + b2_ref[...])


_ROWB = lambda w: pl.BlockSpec((BR, w), lambda i: (i, 0))
_BCAST = lambda r, c: pl.BlockSpec((r, c), lambda i: (0, 0))
_ACCB = pl.BlockSpec((2, BR, DA), lambda i: (0, i, 0))

_PREP_OUT = (
    [jax.ShapeDtypeStruct((N, D), jnp.float32),
     jax.ShapeDtypeStruct((N, DA), jnp.float32),
     jax.ShapeDtypeStruct((N, 8), jnp.float32)],
    [_ROWB(D), _ROWB(DA), _ROWB(8)],
)


def _tc_prep0(x, W1, b1, Wh, bh, A):
    return pl.pallas_call(
        _prep0_body,
        grid=(GRID,),
        in_specs=[_ROWB(D), _BCAST(D, D), _BCAST(1, D), _BCAST(D, D),
                  _BCAST(1, D), _BCAST(D, 8)],
        out_specs=_PREP_OUT[1],
        out_shape=_PREP_OUT[0],
    )(x, W1, b1, Wh, bh, A)


def _tc_combine_prep(acc, y, sc, Wh, bh, A):
    return pl.pallas_call(
        _combine_prep_body,
        grid=(GRID,),
        in_specs=[_ACCB, _ROWB(D), _ROWB(8), _BCAST(D, D), _BCAST(1, D),
                  _BCAST(D, 8)],
        out_specs=_PREP_OUT[1],
        out_shape=_PREP_OUT[0],
    )(acc, y, sc, Wh, bh, A)


def _tc_final(acc, y, sc, W2, b2):
    return pl.pallas_call(
        _final_body,
        grid=(GRID,),
        in_specs=[_ACCB, _ROWB(D), _ROWB(8), _BCAST(D, DOUT),
                  _BCAST(1, DOUT)],
        out_specs=_ROWB(DOUT),
        out_shape=jax.ShapeDtypeStruct((N, DOUT), jnp.float32),
    )(acc, y, sc, W2, b2)


def _sc_body(s_hbm, t_hbm, haug_hbm, x1_hbm, h1_hbm, out_hbm,
             s0, t0, w0, xs0, hs0, rows0, s1, t1, w1, xs1, hs1, rows1,
             st_loc, tt_loc, wt_loc, acc, sem0, sem1):
    c = lax.axis_index("c")
    s = lax.axis_index("s")
    wid = s * 2 + c
    s_loc = (s0, s1)
    t_loc = (t0, t1)
    w_loc = (w0, w1)
    xs_loc = (xs0, xs1)
    hs_loc = (hs0, hs1)
    rows = (rows0, rows1)
    sem = (sem0, sem1)

    @pl.loop(0, C)
    def _zero(e):
        for k in range(DA // 16):
            rows0[e, pl.ds(k * 16, 16)] = jnp.zeros((16,), jnp.float32)

    row0 = s * RPT
    for m in range(RPT // ZCH):
        pltpu.sync_copy(rows0.at[pl.ds(0, ZCH)],
                        acc.at[pl.ds(row0 + m * ZCH, ZCH)])

    @pl.when(s == 15)
    def _zero_rem():
        pltpu.sync_copy(rows0.at[pl.ds(0, REM)], acc.at[pl.ds(RPT * 16, REM)])

    plsc.subcore_barrier()

    base_w = wid * EPW

    def stage_idx(j, b):
        pltpu.sync_copy(s_hbm.at[pl.ds(base_w + j * C, C)], s_loc[b])
        pltpu.sync_copy(t_hbm.at[pl.ds(base_w + j * C, C)], t_loc[b])
        pltpu.async_copy(haug_hbm.at[t_loc[b]], rows[b], sem[b])
        pltpu.async_copy(x1_hbm.at[s_loc[b]], xs_loc[b], sem[b])
        pltpu.async_copy(h1_hbm.at[t_loc[b]], hs_loc[b], sem[b])

    def process(b, nrows):
        pltpu.make_async_copy(haug_hbm.at[t_loc[b]], rows[b], sem[b]).wait()
        pltpu.make_async_copy(x1_hbm.at[s_loc[b]], xs_loc[b], sem[b]).wait()
        pltpu.make_async_copy(h1_hbm.at[t_loc[b]], hs_loc[b], sem[b]).wait()
        for g in range(nrows // 16):
            sl16 = pl.ds(g * 16, 16)
            z = xs_loc[b][sl16] + hs_loc[b][sl16]
            w_loc[b][sl16] = jnp.exp(_leaky(z))

        @pl.loop(0, nrows, unroll=4)
        def _scale(e):
            wv = plsc.load_gather(w_loc[b], [jnp.full((16,), e, jnp.int32)])
            for k in range(DA // 16):
                sl = pl.ds(k * 16, 16)
                rows[b][e, sl] = rows[b][e, sl] * wv

    stage_idx(0, 0)
    stage_idx(1, 1)

    def pair_body(p, carry):
        for b in (0, 1):
            process(b, C)
            pltpu.sync_copy(rows[b], acc.at[s_loc[b]], add=True)

            @pl.when(p < CHW // 2 - 1)
            def _prefetch():
                stage_idx(2 * p + b + 2, b)
        return carry

    lax.fori_loop(0, CHW // 2, pair_body, 0)

    tb = base_w + CHW * C
    pltpu.sync_copy(s_hbm.at[pl.ds(tb, TAIL)], st_loc)
    pltpu.sync_copy(t_hbm.at[pl.ds(tb, TAIL)], tt_loc)
    pltpu.async_copy(haug_hbm.at[tt_loc], rows0.at[pl.ds(0, TAIL)], sem0)
    pltpu.async_copy(x1_hbm.at[st_loc], xs0.at[pl.ds(0, TAIL)], sem0)
    pltpu.async_copy(h1_hbm.at[tt_loc], hs0.at[pl.ds(0, TAIL)], sem0)
    pltpu.make_async_copy(haug_hbm.at[tt_loc], rows0.at[pl.ds(0, TAIL)], sem0).wait()
    pltpu.make_async_copy(x1_hbm.at[st_loc], xs0.at[pl.ds(0, TAIL)], sem0).wait()
    pltpu.make_async_copy(h1_hbm.at[tt_loc], hs0.at[pl.ds(0, TAIL)], sem0).wait()
    zt = xs0[pl.ds(0, TAIL)] + hs0[pl.ds(0, TAIL)]
    wt_loc[...] = jnp.exp(_leaky(zt))

    @pl.loop(0, TAIL)
    def _scale_tail(e):
        wv = plsc.load_gather(wt_loc, [jnp.full((16,), e, jnp.int32)])
        for k in range(DA // 16):
            sl = pl.ds(k * 16, 16)
            rows0[e, sl] = rows0[e, sl] * wv

    pltpu.sync_copy(rows0.at[pl.ds(0, TAIL)], acc.at[st_loc], add=True)
    plsc.subcore_barrier()
    for m in range(RPT // ZCH):
        sl = pl.ds(row0 + m * ZCH, ZCH)
        pltpu.sync_copy(acc.at[sl], out_hbm.at[c, sl])

    @pl.when(s == 15)
    def _out_rem():
        sl = pl.ds(RPT * 16, REM)
        pltpu.sync_copy(acc.at[sl], out_hbm.at[c, sl])


def _sc_edge_pass(s_idx, t_idx, haug, x1, h1):
    mesh = plsc.VectorSubcoreMesh(core_axis_name="c", subcore_axis_name="s")
    buf = [
        pltpu.VMEM((C,), jnp.int32),
        pltpu.VMEM((C,), jnp.int32),
        pltpu.VMEM((C,), jnp.float32),
        pltpu.VMEM((C,), jnp.float32),
        pltpu.VMEM((C,), jnp.float32),
        pltpu.VMEM((C, DA), jnp.float32),
    ]
    return pl.kernel(
        _sc_body,
        out_type=jax.ShapeDtypeStruct((2, N, DA), jnp.float32),
        mesh=mesh,
        compiler_params=pltpu.CompilerParams(needs_layout_passes=False,
                                             use_tc_tiling_on_sc=False),
        scratch_types=buf + buf + [
            pltpu.VMEM((TAIL,), jnp.int32),
            pltpu.VMEM((TAIL,), jnp.int32),
            pltpu.VMEM((TAIL,), jnp.float32),
            pltpu.VMEM_SHARED((N, DA), jnp.float32),
            pltpu.SemaphoreType.DMA,
            pltpu.SemaphoreType.DMA,
        ],
    )(s_idx, t_idx, haug, x1, h1)


def _amat(a1v, a2v, a2o):
    A = jnp.zeros((D, 8), jnp.float32)
    return A.at[:, 0].set(a1v).at[:, 1].set(a2v).at[:, 2].set(a2o)


def kernel(x_user, x_item, edge_index_ui, edge_index_iu, W1_user, b1_user,
           W1_item, b1_item, Wh, bh, a1, a2, W2, b2):
    su = edge_index_ui[0].astype(jnp.int32)
    tu = edge_index_ui[1].astype(jnp.int32)
    si = edge_index_iu[0].astype(jnp.int32)
    ti = edge_index_iu[1].astype(jnp.int32)

    b1u = b1_user.reshape(1, D)
    b1i = b1_item.reshape(1, D)
    bh0 = bh[0].reshape(1, D)
    bh1 = bh[1].reshape(1, D)
    b2r = b2.reshape(1, DOUT)

    A_u0 = _amat(a1[0, 0], a2[0, 0], a2[0, 1])
    A_i0 = _amat(a1[0, 1], a2[0, 1], a2[0, 0])
    A_u1 = _amat(a1[1, 0], a2[1, 0], a2[1, 1])
    A_i1 = _amat(a1[1, 1], a2[1, 1], a2[1, 0])

    yu0, haug_u0, scu0 = _tc_prep0(x_user, W1_user, b1u, Wh[0], bh0, A_u0)
    yi0, haug_i0, sci0 = _tc_prep0(x_item, W1_item, b1i, Wh[0], bh0, A_i0)

    acc_u0 = _sc_edge_pass(su, tu, haug_i0, jnp.copy(scu0[:, 0]),
                           jnp.copy(sci0[:, 2]))
    acc_i0 = _sc_edge_pass(si, ti, haug_u0, jnp.copy(sci0[:, 0]),
                           jnp.copy(scu0[:, 2]))

    yu1, _, scu1 = _tc_combine_prep(acc_u0, yu0, scu0, Wh[1], bh1, A_u1)
    yi1, haug_i1, sci1 = _tc_combine_prep(acc_i0, yi0, sci0, Wh[1], bh1, A_i1)

    acc_u1 = _sc_edge_pass(su, tu, haug_i1, jnp.copy(scu1[:, 0]),
                           jnp.copy(sci1[:, 2]))

    return _tc_final(acc_u1, yu1, scu1, W2, b2r)
